# Initial kernel scaffold; baseline (speedup 1.0000x reference)
import jax
import jax.numpy as jnp
from jax.experimental import pallas as pl


def kernel(x, edge_index, W1, b1, W2, b2, W3, b3):
    raise NotImplementedError("write your pallas kernel here")


# speedup vs baseline: 4.3148x; 4.3148x over previous
import functools

import jax
import jax.numpy as jnp
from jax import lax
from jax.experimental import pallas as pl
from jax.experimental.pallas import tpu as pltpu
from jax.experimental.pallas import tpu_sc as plsc

N = 10000
NP = 10240
E = 320000
F = 128
NCORE = 2
NSUB = 16
CW = 16
DW = 128
C = 80
RPT = NP // NSUB

_mesh = lambda: plsc.VectorSubcoreMesh(
    core_axis_name="c", subcore_axis_name="s", num_cores=NCORE, num_subcores=NSUB)


def _degrees(src, dst, ones2d, zcnt):
    ept = E // NSUB
    nchunks = ept // C

    @functools.partial(
        pl.kernel,
        mesh=_mesh(),
        out_type=jax.ShapeDtypeStruct((NCORE, NP, DW), jnp.float32),
        scratch_types=[
            pltpu.VMEM((C,), jnp.int32),
            pltpu.VMEM((C, DW), jnp.float32),
            pltpu.VMEM_SHARED((NP, DW), jnp.float32),
        ],
    )
    def k(src_hbm, dst_hbm, ones_hbm, z_hbm, cnt_hbm, idx_v, ones_v, acc):
        c = lax.axis_index("c")
        s = lax.axis_index("s")
        pltpu.sync_copy(ones_hbm, ones_v)
        pltpu.sync_copy(z_hbm, acc.at[pl.ds(s * RPT, RPT)])
        plsc.subcore_barrier()

        def run(e_hbm):
            def chunk(i, carry):
                eb = s * ept + i * C
                pltpu.sync_copy(e_hbm.at[pl.ds(eb, C)], idx_v)
                pltpu.sync_copy(ones_v, acc.at[idx_v], add=True)
                return carry

            lax.fori_loop(0, nchunks, chunk, 0)

        @pl.when(c == 0)
        def _():
            run(src_hbm)

        @pl.when(c == 1)
        def _():
            run(dst_hbm)

        plsc.subcore_barrier()
        pltpu.sync_copy(acc.at[pl.ds(s * RPT, RPT)],
                        cnt_hbm.at[c, pl.ds(s * RPT, RPT)])

    return k(src, dst, ones2d, zcnt)


def _aggregate(h0, h1, src, dst, zrows, feat_split):
    if feat_split:
        ept = E // NSUB
    else:
        ept = E // (NCORE * NSUB)
    nchunks = ept // C

    @functools.partial(
        pl.kernel,
        mesh=_mesh(),
        out_type=[jax.ShapeDtypeStruct((NP, F), jnp.float32),
                  jax.ShapeDtypeStruct((NP, F), jnp.float32)],
        scratch_types=[
            pltpu.VMEM((C,), jnp.int32),
            pltpu.VMEM((C,), jnp.int32),
            pltpu.VMEM((C, F), jnp.float32),
            pltpu.VMEM_SHARED((NP, F), jnp.float32),
            pltpu.SemaphoreType.DMA,
        ],
    )
    def k(h0_hbm, h1_hbm, src_hbm, dst_hbm, z_hbm, o0_hbm, o1_hbm,
          idx_s, idx_d, rows, acc, sem):
        c = lax.axis_index("c")
        s = lax.axis_index("s")
        pltpu.sync_copy(z_hbm, acc.at[pl.ds(s * RPT, RPT)])
        plsc.subcore_barrier()

        def run(h_hbm, out_hbm, base0):
            def chunk(i, carry):
                eb = base0 + i * C
                pltpu.sync_copy(src_hbm.at[pl.ds(eb, C)], idx_s)
                pltpu.sync_copy(dst_hbm.at[pl.ds(eb, C)], idx_d)
                pltpu.async_copy(h_hbm.at[idx_s], rows, sem).wait()
                pltpu.sync_copy(rows, acc.at[idx_d], add=True)
                return carry

            lax.fori_loop(0, nchunks, chunk, 0)
            plsc.subcore_barrier()
            pltpu.sync_copy(acc.at[pl.ds(s * RPT, RPT)],
                            out_hbm.at[pl.ds(s * RPT, RPT)])

        if feat_split:
            @pl.when(c == 0)
            def _():
                run(h0_hbm, o0_hbm, s * ept)

            @pl.when(c == 1)
            def _():
                run(h1_hbm, o1_hbm, s * ept)
        else:
            @pl.when(c == 0)
            def _():
                run(h0_hbm, o0_hbm, s * ept)

            @pl.when(c == 1)
            def _():
                run(h1_hbm, o1_hbm, (NSUB + s) * ept)

    return k(h0, h1, src, dst, zrows)


_BR = 1024


def _prep(cnt, x):
    grid = NP // _BR

    def body(cnt_ref, x_ref, ns_ref, nd_ref, xs_ref):
        cs = cnt_ref[0, :, :CW]
        cd = cnt_ref[1, :, :CW]
        ns = jnp.where(cs > 0, lax.rsqrt(jnp.maximum(cs, 1.0)), 0.0)
        nd = jnp.where(cd > 0, lax.rsqrt(jnp.maximum(cd, 1.0)), 0.0)
        ns_ref[...] = ns
        nd_ref[...] = nd
        xs_ref[...] = x_ref[...] * ns[:, :1]

    return pl.pallas_call(
        body,
        grid=(grid,),
        in_specs=[
            pl.BlockSpec((NCORE, _BR, DW), lambda i: (0, i, 0)),
            pl.BlockSpec((_BR, 128), lambda i: (i, 0)),
        ],
        out_specs=[
            pl.BlockSpec((_BR, CW), lambda i: (i, 0)),
            pl.BlockSpec((_BR, CW), lambda i: (i, 0)),
            pl.BlockSpec((_BR, 128), lambda i: (i, 0)),
        ],
        out_shape=[
            jax.ShapeDtypeStruct((NP, CW), jnp.float32),
            jax.ShapeDtypeStruct((NP, CW), jnp.float32),
            jax.ShapeDtypeStruct((NP, 128), jnp.float32),
        ],
    )(cnt, x)


def _mm1(p0, p1, nd, ns, W1, b1):
    grid = NP // _BR

    def body(p0_ref, p1_ref, nd_ref, ns_ref, w_ref, b_ref, y0_ref, y1_ref):
        agg = (p0_ref[...] + p1_ref[...]) * nd_ref[...][:, :1]
        h = jnp.dot(agg, w_ref[...], preferred_element_type=jnp.float32)
        h = jnp.maximum(h + b_ref[...], 0.0) * ns_ref[...][:, :1]
        y0_ref[...] = h[:, :128]
        y1_ref[...] = h[:, 128:]

    return pl.pallas_call(
        body,
        grid=(grid,),
        in_specs=[
            pl.BlockSpec((_BR, 128), lambda i: (i, 0)),
            pl.BlockSpec((_BR, 128), lambda i: (i, 0)),
            pl.BlockSpec((_BR, CW), lambda i: (i, 0)),
            pl.BlockSpec((_BR, CW), lambda i: (i, 0)),
            pl.BlockSpec((128, 256), lambda i: (0, 0)),
            pl.BlockSpec((1, 256), lambda i: (0, 0)),
        ],
        out_specs=[
            pl.BlockSpec((_BR, 128), lambda i: (i, 0)),
            pl.BlockSpec((_BR, 128), lambda i: (i, 0)),
        ],
        out_shape=[
            jax.ShapeDtypeStruct((NP, 128), jnp.float32),
            jax.ShapeDtypeStruct((NP, 128), jnp.float32),
        ],
    )(p0, p1, nd, ns, W1, b1)


def _mm2(a0, a1, nd, ns, W2, b2, W3):
    grid = NP // _BR

    def body(a0_ref, a1_ref, nd_ref, ns_ref, w2a_ref, w2b_ref, b_ref, w3_ref,
             t_ref):
        ndb = nd_ref[...][:, :1]
        h = jnp.dot(a0_ref[...] * ndb, w2a_ref[...],
                    preferred_element_type=jnp.float32)
        h += jnp.dot(a1_ref[...] * ndb, w2b_ref[...],
                     preferred_element_type=jnp.float32)
        h = jnp.maximum(h + b_ref[...], 0.0) * ns_ref[...][:, :1]
        t_ref[...] = jnp.dot(h, w3_ref[...], preferred_element_type=jnp.float32)

    return pl.pallas_call(
        body,
        grid=(grid,),
        in_specs=[
            pl.BlockSpec((_BR, 128), lambda i: (i, 0)),
            pl.BlockSpec((_BR, 128), lambda i: (i, 0)),
            pl.BlockSpec((_BR, CW), lambda i: (i, 0)),
            pl.BlockSpec((_BR, CW), lambda i: (i, 0)),
            pl.BlockSpec((128, 256), lambda i: (0, 0)),
            pl.BlockSpec((128, 256), lambda i: (0, 0)),
            pl.BlockSpec((1, 256), lambda i: (0, 0)),
            pl.BlockSpec((256, 128), lambda i: (0, 0)),
        ],
        out_specs=pl.BlockSpec((_BR, 128), lambda i: (i, 0)),
        out_shape=jax.ShapeDtypeStruct((NP, 128), jnp.float32),
    )(a0, a1, nd, ns, W2[:128], W2[128:], b2, W3)


def _final(q0, q1, nd, b3):
    grid = NP // _BR

    def body(q0_ref, q1_ref, nd_ref, b_ref, o_ref):
        agg = (q0_ref[...] + q1_ref[...]) * nd_ref[...][:, :1]
        o_ref[...] = jnp.maximum(agg + b_ref[...], 0.0)

    return pl.pallas_call(
        body,
        grid=(grid,),
        in_specs=[
            pl.BlockSpec((_BR, 128), lambda i: (i, 0)),
            pl.BlockSpec((_BR, 128), lambda i: (i, 0)),
            pl.BlockSpec((_BR, CW), lambda i: (i, 0)),
            pl.BlockSpec((1, 128), lambda i: (0, 0)),
        ],
        out_specs=pl.BlockSpec((_BR, 128), lambda i: (i, 0)),
        out_shape=jax.ShapeDtypeStruct((NP, 128), jnp.float32),
    )(q0, q1, nd, b3)


def kernel(x, edge_index, W1, b1, W2, b2, W3, b3):
    eidx = edge_index.astype(jnp.int32)
    src = eidx[0]
    dst = eidx[1]
    xp = jnp.pad(x, ((0, NP - N), (0, 0)))
    ones2d = jnp.ones((C, DW), jnp.float32)
    zcnt = jnp.zeros((RPT, DW), jnp.float32)
    zrows = jnp.zeros((RPT, F), jnp.float32)
    b1r = b1.reshape(1, -1)
    b2r = b2.reshape(1, -1)
    b3r = b3.reshape(1, -1)

    cnt = _degrees(src, dst, ones2d, zcnt)
    ns, nd, xs = _prep(cnt, xp)
    p0, p1 = _aggregate(xs, xs, src, dst, zrows, feat_split=False)
    y0, y1 = _mm1(p0, p1, nd, ns, W1, b1r)
    a0, a1 = _aggregate(y0, y1, src, dst, zrows, feat_split=True)
    t = _mm2(a0, a1, nd, ns, W2, b2r, W3)
    q0, q1 = _aggregate(t, t, src, dst, zrows, feat_split=False)
    return _final(q0, q1, nd, b3r)[:N]
